# baseline (speedup 1.0000x reference)
import functools

import jax
import jax.numpy as jnp
from jax import lax
from jax.experimental import pallas as pl
from jax.experimental.pallas import tpu as pltpu
from jax.experimental.pallas import tpu_sc as plsc

_NC = 2
_NS = 16
_NW = _NC * _NS
_L = 16

_CHUNK_NODES = 1024


def _pool_kernel(n_pairs, n, nodes, nodes_pad):
    pairs_per_w = n_pairs // _NW
    n_vec = nodes_pad // _L
    n_chunks = nodes // _CHUNK_NODES
    tail_nodes = nodes - n_chunks * _CHUNK_NODES
    chunk_words = 7 * _CHUNK_NODES
    mesh = plsc.VectorSubcoreMesh(core_axis_name="c", subcore_axis_name="s")

    @functools.partial(
        pl.kernel,
        mesh=mesh,
        compiler_params=pltpu.CompilerParams(
            needs_layout_passes=False, use_tc_tiling_on_sc=False
        ),
        out_type=jax.ShapeDtypeStruct((n_pairs, nodes), jnp.float32),
        scratch_types=[
            pltpu.VMEM((n,), jnp.float32),
            pltpu.VMEM((7 * nodes_pad,), jnp.int32),
            pltpu.VMEM((nodes_pad,), jnp.float32),
            pltpu.VMEM((chunk_words,), jnp.int32),
        ],
    )
    def body(x_hbm, neigh_hbm, out_hbm, xrow, idxv, outv, chunkv):
        wid = lax.axis_index("s") * _NC + lax.axis_index("c")
        base = wid * pairs_per_w
        lane7 = lax.iota(jnp.int32, _L) * 7

        def per_chunk(c, carry):
            pltpu.sync_copy(neigh_hbm.at[pl.ds(c * chunk_words, chunk_words)],
                            chunkv)

            def per_cvec(jv, carry2):
                node_off = c * _CHUNK_NODES + jv * _L
                for k in range(7):
                    src = lane7 + (jv * (7 * _L) + k)
                    vals = plsc.load_gather(chunkv, [src])
                    idxv[pl.ds(k * nodes_pad + node_off, _L)] = vals
                return carry2

            lax.fori_loop(0, _CHUNK_NODES // _L, per_cvec, 0, unroll=False)
            return carry

        lax.fori_loop(0, n_chunks, per_chunk, 0, unroll=False)

        if tail_nodes:
            tail_words = 7 * tail_nodes
            pltpu.sync_copy(
                neigh_hbm.at[pl.ds(n_chunks * chunk_words, tail_words)],
                chunkv.at[pl.ds(0, tail_words)],
            )
            node_off = n_chunks * _CHUNK_NODES
            for k in range(7):
                src = jnp.minimum(lane7 + k, tail_words - 1)
                vals = plsc.load_gather(chunkv, [src])
                idxv[pl.ds(k * nodes_pad + node_off, _L)] = vals

        def per_pair(p, carry):
            pair = base + p
            pltpu.sync_copy(x_hbm.at[pair], xrow)

            def per_vec(j, carry2):
                off = j * _L
                acc = jnp.zeros((_L,), jnp.float32)
                for k in range(7):
                    iv = idxv[pl.ds(k * nodes_pad + off, _L)]
                    acc = acc + plsc.load_gather(xrow, [iv])
                outv[pl.ds(off, _L)] = acc * jnp.float32(1.0 / 7.0)
                return carry2

            lax.fori_loop(0, n_vec, per_vec, 0, unroll=False)
            pltpu.sync_copy(outv.at[pl.ds(0, nodes)], out_hbm.at[pair])
            return carry

        lax.fori_loop(0, pairs_per_w, per_pair, 0, unroll=False)

    return body


def kernel(x, neigh_orders):
    B, C, N = x.shape
    nodes = (N + 6) // 4
    nodes_pad = ((nodes + _L - 1) // _L) * _L
    n_pairs = B * C

    x2 = x.reshape(n_pairs, N)
    out = _pool_kernel(n_pairs, N, nodes, nodes_pad)(x2, neigh_orders)
    return out.reshape(B, C, nodes)


# speedup vs baseline: 1.0704x; 1.0704x over previous
import functools

import jax
import jax.numpy as jnp
from jax import lax
from jax.experimental import pallas as pl
from jax.experimental.pallas import tpu as pltpu
from jax.experimental.pallas import tpu_sc as plsc

_NC = 2
_NS = 16
_NW = _NC * _NS
_L = 16

_CHUNK_NODES = 1024


def _pool_kernel(n_pairs, n, nodes, nodes_pad):
    pairs_per_w = n_pairs // _NW
    n_vec = nodes_pad // _L
    n_chunks = nodes // _CHUNK_NODES
    tail_nodes = nodes - n_chunks * _CHUNK_NODES
    chunk_words = 7 * _CHUNK_NODES
    mesh = plsc.VectorSubcoreMesh(core_axis_name="c", subcore_axis_name="s")

    row_copy = ((n + 6) // 8) * 8

    @functools.partial(
        pl.kernel,
        mesh=mesh,
        compiler_params=pltpu.CompilerParams(
            needs_layout_passes=False, use_tc_tiling_on_sc=False
        ),
        out_type=jax.ShapeDtypeStruct((n_pairs, nodes), jnp.float32),
        scratch_types=[
            pltpu.VMEM((row_copy,), jnp.float32),
            pltpu.VMEM((7 * nodes_pad,), jnp.int32),
            pltpu.VMEM((nodes_pad,), jnp.float32),
            pltpu.VMEM((chunk_words,), jnp.int32),
        ],
    )
    def body(x_hbm, neigh_hbm, out_hbm, xrow, idxv, outv, chunkv):
        wid = lax.axis_index("s") * _NC + lax.axis_index("c")
        base = wid * pairs_per_w
        lane7 = lax.iota(jnp.int32, _L) * 7

        def per_chunk(c, carry):
            pltpu.sync_copy(neigh_hbm.at[pl.ds(c * chunk_words, chunk_words)],
                            chunkv)

            def per_cvec(jv, carry2):
                node_off = c * _CHUNK_NODES + jv * _L
                for k in range(7):
                    src = lane7 + (jv * (7 * _L) + k)
                    vals = plsc.load_gather(chunkv, [src])
                    idxv[pl.ds(k * nodes_pad + node_off, _L)] = vals
                return carry2

            lax.fori_loop(0, _CHUNK_NODES // _L, per_cvec, 0, unroll=False)
            return carry

        lax.fori_loop(0, n_chunks, per_chunk, 0, unroll=False)

        if tail_nodes:
            tail_words = 7 * tail_nodes
            pltpu.sync_copy(
                neigh_hbm.at[pl.ds(n_chunks * chunk_words, tail_words)],
                chunkv.at[pl.ds(0, tail_words)],
            )
            node_off = n_chunks * _CHUNK_NODES
            for k in range(7):
                src = jnp.minimum(lane7 + k, tail_words - 1)
                vals = plsc.load_gather(chunkv, [src])
                idxv[pl.ds(k * nodes_pad + node_off, _L)] = vals

        def per_pair(p, carry):
            pair = base + p
            start = pair * n
            r = lax.rem(start, 8)
            astart = pl.multiple_of(start - r, 8)
            pltpu.sync_copy(x_hbm.at[pl.ds(astart, row_copy)], xrow)

            def per_vec(j, carry2):
                off = j * _L
                acc = jnp.zeros((_L,), jnp.float32)
                for k in range(7):
                    iv = idxv[pl.ds(k * nodes_pad + off, _L)] + r
                    acc = acc + plsc.load_gather(xrow, [iv])
                outv[pl.ds(off, _L)] = acc * jnp.float32(1.0 / 7.0)
                return carry2

            lax.fori_loop(0, n_vec, per_vec, 0, unroll=2)
            pltpu.sync_copy(outv.at[pl.ds(0, nodes)], out_hbm.at[pair])
            return carry

        lax.fori_loop(0, pairs_per_w, per_pair, 0, unroll=False)

    return body


def kernel(x, neigh_orders):
    B, C, N = x.shape
    nodes = (N + 6) // 4
    nodes_pad = ((nodes + _L - 1) // _L) * _L
    n_pairs = B * C

    xf = x.reshape(n_pairs * N)
    out = _pool_kernel(n_pairs, N, nodes, nodes_pad)(xf, neigh_orders)
    return out.reshape(B, C, nodes)


# speedup vs baseline: 10.5912x; 9.8946x over previous
import functools

import jax
import jax.numpy as jnp
from jax import lax
from jax.experimental import pallas as pl
from jax.experimental.pallas import tpu as pltpu
from jax.experimental.pallas import tpu_sc as plsc

_NC = 2
_NS = 16
_NW = _NC * _NS
_L = 16

_CN = 16


def _pool_kernel(n, nodes, d):
    n_chunks = nodes // _CN
    tail_nodes = nodes - n_chunks * _CN
    chunks_per_w = n_chunks // _NW
    assert chunks_per_w * _NW == n_chunks
    widx_words = chunks_per_w * _CN * 7
    d_vec = d // _L
    mesh = plsc.VectorSubcoreMesh(core_axis_name="c", subcore_axis_name="s")

    @functools.partial(
        pl.kernel,
        mesh=mesh,
        compiler_params=pltpu.CompilerParams(
            needs_layout_passes=False, use_tc_tiling_on_sc=False
        ),
        out_type=jax.ShapeDtypeStruct((nodes, d), jnp.float32),
        scratch_types=[
            pltpu.VMEM((7 * _CN, d), jnp.float32),
            pltpu.VMEM((widx_words,), jnp.int32),
            pltpu.SemaphoreType.DMA,
        ],
    )
    def body(x_hbm, neigh_hbm, out_hbm, gbuf, rawidx, sem):
        wid = lax.axis_index("s") * _NC + lax.axis_index("c")
        inv7 = jnp.float32(1.0 / 7.0)

        pltpu.sync_copy(neigh_hbm.at[pl.ds(wid * widx_words, widx_words)],
                        rawidx)

        def reduce_rows(n_out):
            for j in range(n_out):
                def per_c(ci, carry):
                    off = ci * _L
                    acc = gbuf[7 * j, pl.ds(off, _L)]
                    for k in range(1, 7):
                        acc = acc + gbuf[7 * j + k, pl.ds(off, _L)]
                    gbuf[j, pl.ds(off, _L)] = acc * inv7
                    return carry

                lax.fori_loop(0, d_vec, per_c, 0, unroll=False)

        def per_chunk(gl, carry):
            node0 = (wid * chunks_per_w + gl) * _CN
            pltpu.async_copy(
                x_hbm.at[rawidx.at[pl.ds(gl * (7 * _CN), 7 * _CN)]],
                gbuf, sem,
            ).wait()
            reduce_rows(_CN)
            pltpu.sync_copy(gbuf.at[pl.ds(0, _CN)],
                            out_hbm.at[pl.ds(node0, _CN)])
            return carry

        lax.fori_loop(0, chunks_per_w, per_chunk, 0, unroll=False)

        if tail_nodes:
            tail_words = 7 * tail_nodes

            @pl.when(wid == 0)
            def _():
                pltpu.sync_copy(
                    neigh_hbm.at[pl.ds(n_chunks * _CN * 7, tail_words)],
                    rawidx.at[pl.ds(0, tail_words)],
                )
                pltpu.async_copy(
                    x_hbm.at[rawidx.at[pl.ds(0, tail_words)]],
                    gbuf.at[pl.ds(0, tail_words)], sem,
                ).wait()
                reduce_rows(tail_nodes)
                pltpu.sync_copy(gbuf.at[pl.ds(0, tail_nodes)],
                                out_hbm.at[pl.ds(n_chunks * _CN, tail_nodes)])

    return body


def kernel(x, neigh_orders):
    B, C, N = x.shape
    nodes = (N + 6) // 4
    d = B * C

    xt = jnp.transpose(x, (2, 0, 1)).reshape(N, d)
    out = _pool_kernel(N, nodes, d)(xt, neigh_orders)
    return jnp.transpose(out.reshape(nodes, B, C), (1, 2, 0))


# speedup vs baseline: 17.8711x; 1.6873x over previous
import functools

import jax
import jax.numpy as jnp
from jax import lax
from jax.experimental import pallas as pl
from jax.experimental.pallas import tpu as pltpu
from jax.experimental.pallas import tpu_sc as plsc

_NC = 2
_NS = 16
_NW = _NC * _NS
_L = 16

_CN = 8


def _pool_kernel(n, nodes, d):
    n_chunks = nodes // _CN
    tail_nodes = nodes - n_chunks * _CN
    chunks_per_w = n_chunks // _NW
    assert chunks_per_w * _NW == n_chunks and chunks_per_w % 2 == 0
    cw = 7 * _CN
    widx_words = chunks_per_w * cw
    d_vec = d // _L
    mesh = plsc.VectorSubcoreMesh(core_axis_name="c", subcore_axis_name="s")

    @functools.partial(
        pl.kernel,
        mesh=mesh,
        compiler_params=pltpu.CompilerParams(
            needs_layout_passes=False, use_tc_tiling_on_sc=False
        ),
        out_type=jax.ShapeDtypeStruct((nodes, d), jnp.float32),
        scratch_types=[
            pltpu.VMEM((cw, d), jnp.float32),
            pltpu.VMEM((cw, d), jnp.float32),
            pltpu.VMEM((widx_words,), jnp.int32),
            pltpu.SemaphoreType.DMA,
            pltpu.SemaphoreType.DMA,
            pltpu.SemaphoreType.DMA,
            pltpu.SemaphoreType.DMA,
        ],
    )
    def body(x_hbm, neigh_hbm, out_hbm, buf0, buf1, rawidx,
             gsem0, gsem1, osem0, osem1):
        wid = lax.axis_index("s") * _NC + lax.axis_index("c")
        inv7 = jnp.float32(1.0 / 7.0)
        bufs = (buf0, buf1)
        gsems = (gsem0, gsem1)
        osems = (osem0, osem1)
        base = wid * chunks_per_w

        pltpu.sync_copy(neigh_hbm.at[pl.ds(wid * widx_words, widx_words)],
                        rawidx)

        def gather_src(g):
            return x_hbm.at[rawidx.at[pl.ds(g * cw, cw)]]

        def start_gather(g, b):
            pltpu.make_async_copy(gather_src(g), bufs[b], gsems[b]).start()

        def reduce_rows(buf, n_out):
            def per_c(ci, carry):
                off = ci * _L
                for j in range(n_out):
                    acc = buf[7 * j, pl.ds(off, _L)]
                    for k in range(1, 7):
                        acc = acc + buf[7 * j + k, pl.ds(off, _L)]
                    buf[j, pl.ds(off, _L)] = acc * inv7
                return carry

            lax.fori_loop(0, d_vec, per_c, 0, unroll=False)

        start_gather(0, 0)

        def per_iter(i, carry):
            for b in range(2):
                g = i * 2 + b
                nb = 1 - b
                @pl.when(g + 1 < chunks_per_w)
                def _():
                    @pl.when(g >= 1)
                    def _():
                        pltpu.make_async_copy(
                            bufs[nb].at[pl.ds(0, _CN)],
                            out_hbm.at[pl.ds((base + g) * _CN, _CN)],
                            osems[nb],
                        ).wait()
                    start_gather(g + 1, nb)
                pltpu.make_async_copy(gather_src(g), bufs[b], gsems[b]).wait()
                reduce_rows(bufs[b], _CN)
                pltpu.make_async_copy(
                    bufs[b].at[pl.ds(0, _CN)],
                    out_hbm.at[pl.ds((base + g) * _CN, _CN)],
                    osems[b],
                ).start()
            return carry

        lax.fori_loop(0, chunks_per_w // 2, per_iter, 0, unroll=False)

        for b in range(2):
            pltpu.make_async_copy(
                bufs[b].at[pl.ds(0, _CN)],
                out_hbm.at[pl.ds(base * _CN, _CN)],
                osems[b],
            ).wait()

        if tail_nodes:
            tail_words = 7 * tail_nodes

            @pl.when(wid == 0)
            def _():
                pltpu.sync_copy(
                    neigh_hbm.at[pl.ds(n_chunks * cw, tail_words)],
                    rawidx.at[pl.ds(0, tail_words)],
                )
                pltpu.async_copy(
                    x_hbm.at[rawidx.at[pl.ds(0, tail_words)]],
                    buf0.at[pl.ds(0, tail_words)], gsem0,
                ).wait()
                reduce_rows(buf0, tail_nodes)
                pltpu.sync_copy(buf0.at[pl.ds(0, tail_nodes)],
                                out_hbm.at[pl.ds(n_chunks * _CN, tail_nodes)])

    return body


def kernel(x, neigh_orders):
    B, C, N = x.shape
    nodes = (N + 6) // 4
    d = B * C

    xt = jnp.transpose(x, (2, 0, 1)).reshape(N, d)
    out = _pool_kernel(N, nodes, d)(xt, neigh_orders)
    return jnp.transpose(out.reshape(nodes, B, C), (1, 2, 0))
